# baseline (speedup 1.0000x reference)
import functools

import jax
import jax.numpy as jnp
from jax import lax
from jax.experimental import pallas as pl
from jax.experimental.pallas import tpu as pltpu
from jax.experimental.pallas import tpu_sc as plsc

VOCAB = 1000000
EMBED_DIM = 32
BATCH = 16384
N_FIELDS = 26

TOT = BATCH * N_FIELDS
NUM_CORES = 2
NUM_SUBCORES = 16
NW = NUM_CORES * NUM_SUBCORES
SUB = 128
NBLK = TOT // SUB
BLK_PER_W = NBLK // NW
BC = BATCH // SUB

PITCH = EMBED_DIM
VPITCH = EMBED_DIM + 1
SW = 256
NSTRIPE = VOCAB // SW
TAIL = VOCAB - NSTRIPE * SW
STRIPE_Q, STRIPE_R = divmod(NSTRIPE, NW)

_mesh = plsc.VectorSubcoreMesh(core_axis_name="c", subcore_axis_name="s")


@functools.partial(
    pl.kernel,
    mesh=_mesh,
    out_type=jax.ShapeDtypeStruct((VOCAB * PITCH,), jnp.float32),
    scratch_types=[
        pltpu.VMEM((EMBED_DIM, SW + 1), jnp.float32),
        pltpu.VMEM((EMBED_DIM, SW + 1), jnp.float32),
        pltpu.VMEM((SW * PITCH,), jnp.float32),
        pltpu.VMEM((SW * PITCH,), jnp.float32),
        pltpu.VMEM((TAIL * EMBED_DIM,), jnp.float32),
        pltpu.SemaphoreType.DMA,
        pltpu.SemaphoreType.DMA,
        pltpu.SemaphoreType.DMA,
        pltpu.SemaphoreType.DMA,
    ],
    compiler_params=pltpu.CompilerParams(
        use_tc_tiling_on_sc=True, needs_layout_passes=False
    ),
)
def _table_prep(w4_hbm, wtail_hbm, t_hbm, vbufa, vbufb, trowa, trowb, tailv,
                isema, isemb, osema, osemb):
    wid = lax.axis_index("s") * NUM_CORES + lax.axis_index("c")
    nst = jnp.where(wid < STRIPE_R, STRIPE_Q + 1, STRIPE_Q)
    start = wid * STRIPE_Q + jnp.minimum(wid, STRIPE_R)

    iota = lax.iota(jnp.int32, 16)

    def fire_in(s, vbuf, sem):
        for dq in range(4):
            pltpu.async_copy(
                w4_hbm.at[dq, :, pl.ds(s * SW, SW)],
                vbuf.at[pl.ds(dq * 8, 8), pl.ds(0, SW)],
                sem,
            )

    def drain_in(vbuf, sem):
        for dq in range(4):
            pltpu.make_async_copy(
                w4_hbm.at[0, :, pl.ds(0, SW)],
                vbuf.at[pl.ds(0, 8), pl.ds(0, SW)],
                sem,
            ).wait()

    def transpose(vbuf, trow):
        @plsc.parallel_loop(0, SW, step=4, unroll=8)
        def _vl_body(vl0):
            for u in range(4):
                vl = vl0 + u
                colv = jnp.full((16,), vl, jnp.int32)
                for c in range(EMBED_DIM // 16):
                    vals = plsc.load_gather(vbuf, [iota + c * 16, colv])
                    idx = iota + (vl * PITCH + c * 16)
                    plsc.store_scatter(trow, [idx], vals)

    def fire_out(s, trow, sem):
        pltpu.async_copy(
            trow, t_hbm.at[pl.ds(s * SW * PITCH, SW * PITCH)], sem
        )

    def drain_out(trow, sem):
        pltpu.make_async_copy(
            trow, t_hbm.at[pl.ds(0, SW * PITCH)], sem
        ).wait()

    def step(i, vbuf, trow, isem, osem, first):
        drain_in(vbuf, isem)
        if not first:
            drain_out(trow, osem)
        transpose(vbuf, trow)
        fire_out(start + i, trow, osem)

    fire_in(start, vbufa, isema)
    fire_in(start + 1, vbufb, isemb)

    def half(p, i0, vbuf, trow, isem, osem, first, off):
        step(i0 + off, vbuf, trow, isem, osem, first)

        @pl.when(i0 + off + 2 < nst)
        def _():
            fire_in(start + i0 + off + 2, vbuf, isem)

    half(0, 0, vbufa, trowa, isema, osema, True, 0)
    half(0, 0, vbufb, trowb, isemb, osemb, True, 1)

    def pair_body(p, carry):
        i0 = 2 * p
        half(p, i0, vbufa, trowa, isema, osema, False, 0)
        half(p, i0, vbufb, trowb, isemb, osemb, False, 1)
        return carry

    lax.fori_loop(1, STRIPE_Q // 2, pair_body, 0)

    @pl.when(wid < STRIPE_R)
    def _():
        step(STRIPE_Q, vbufa, trowa, isema, osema, False)

    drain_out(trowa, osema)
    drain_out(trowb, osemb)

    @pl.when(wid == NW - 1)
    def _():
        pltpu.sync_copy(wtail_hbm, tailv)

        def tail_body(r, carry):
            for c in range(EMBED_DIM // 16):
                vals = tailv[pl.ds(r * EMBED_DIM + c * 16, 16)]
                idx = iota + (r * PITCH + c * 16)
                plsc.store_scatter(trowa, [idx], vals)
            return carry

        lax.fori_loop(0, TAIL, tail_body, 0)
        pltpu.sync_copy(
            trowa.at[pl.ds(0, TAIL * PITCH)],
            t_hbm.at[pl.ds(NSTRIPE * SW * PITCH, TAIL * PITCH)],
        )


@functools.partial(
    pl.kernel,
    mesh=_mesh,
    out_type=jax.ShapeDtypeStruct((N_FIELDS, 4, BC, 8 * SUB), jnp.float32),
    scratch_types=[
        pltpu.VMEM((BLK_PER_W, SUB), jnp.int32),
        pltpu.VMEM((SUB, EMBED_DIM), jnp.float32),
        pltpu.VMEM((SUB, EMBED_DIM), jnp.float32),
        pltpu.VMEM((SUB, VPITCH), jnp.float32),
        pltpu.VMEM((SUB, VPITCH), jnp.float32),
        pltpu.VMEM((SUB * EMBED_DIM,), jnp.float32),
        pltpu.VMEM((SUB * EMBED_DIM,), jnp.float32),
        pltpu.SemaphoreType.DMA,
        pltpu.SemaphoreType.DMA,
        pltpu.SemaphoreType.DMA,
        pltpu.SemaphoreType.DMA,
    ],
    compiler_params=pltpu.CompilerParams(
        use_tc_tiling_on_sc=False, needs_layout_passes=False
    ),
)
def _emb_lookup(idx_hbm, table_hbm, out_hbm, idx_v, bufa, bufb, pbufa, pbufb,
                tbufa, tbufb, gsema, gsemb, osema, osemb):
    wid = lax.axis_index("s") * NUM_CORES + lax.axis_index("c")
    base = wid * BLK_PER_W

    pltpu.sync_copy(idx_hbm.at[pl.ds(base, BLK_PER_W)], idx_v)

    iota = lax.iota(jnp.int32, 16)

    def fire_gather(j, buf, sem):
        pltpu.async_copy(table_hbm.at[idx_v.at[j]], buf, sem)

    def drain_gather(buf, sem):
        pltpu.make_async_copy(table_hbm.at[pl.ds(0, SUB)], buf, sem).wait()

    def transpose(buf, pbuf, tbuf):
        @plsc.parallel_loop(0, SUB, step=4, unroll=4)
        def _bl_body(bl0):
            for u in range(4):
                bl = bl0 + u
                for c in range(EMBED_DIM // 16):
                    vals = buf[bl, pl.ds(c * 16, 16)]
                    pbuf[bl, pl.ds(c * 16, 16)] = vals

        @plsc.parallel_loop(0, EMBED_DIM, step=4, unroll=4)
        def _d_body(d0):
            for u in range(4):
                d = d0 + u
                col = jnp.full((16,), d, jnp.int32)
                for c in range(SUB // 16):
                    vals = plsc.load_gather(pbuf, [iota + c * 16, col])
                    tbuf[pl.ds(d * SUB + c * 16, 16)] = vals

    def fire_out(j, tbuf, sem):
        blk = base + j
        f = blk // BC
        bc = blk % BC
        for dq in range(4):
            pltpu.async_copy(
                tbuf.at[pl.ds(dq * 8 * SUB, 8 * SUB)],
                out_hbm.at[f, dq, bc],
                sem,
            )

    def drain_out(tbuf, sem):
        for dq in range(4):
            pltpu.make_async_copy(
                tbuf.at[pl.ds(0, 8 * SUB)], out_hbm.at[0, 0, 0], sem
            ).wait()

    fire_gather(0, bufa, gsema)

    def step(j, buf, pbuf, tbuf, gsem, osem, first, fire_next, nxt_buf,
             nxt_gsem):
        if fire_next:
            fire_gather(j + 1, nxt_buf, nxt_gsem)
        drain_gather(buf, gsem)
        if not first:
            drain_out(tbuf, osem)
        transpose(buf, pbuf, tbuf)
        fire_out(j, tbuf, osem)

    step(0, bufa, pbufa, tbufa, gsema, osema, True, True, bufb, gsemb)
    step(1, bufb, pbufb, tbufb, gsemb, osemb, True, True, bufa, gsema)

    def pair_body(p, carry):
        j0 = 2 * p
        step(j0, bufa, pbufa, tbufa, gsema, osema, False, True, bufb, gsemb)
        step(j0 + 1, bufb, pbufb, tbufb, gsemb, osemb, False, True, bufa,
             gsema)
        return carry

    lax.fori_loop(1, BLK_PER_W // 2 - 1, pair_body, 0)

    step(BLK_PER_W - 2, bufa, pbufa, tbufa, gsema, osema, False, True, bufb,
         gsemb)
    step(BLK_PER_W - 1, bufb, pbufb, tbufb, gsemb, osemb, False, False, bufa,
         gsema)

    drain_out(tbufa, osema)
    drain_out(tbufb, osemb)


def kernel(input, weight):
    w4 = jnp.transpose(weight).reshape(4, 8, VOCAB)
    wtail = weight[NSTRIPE * SW:].reshape(TAIL * EMBED_DIM)
    t1d = _table_prep(w4, wtail)
    table2 = t1d.reshape(VOCAB, PITCH)

    idx2d = jnp.transpose(input).reshape(NBLK, SUB)
    out4 = _emb_lookup(idx2d, table2)
    out5 = out4.reshape(N_FIELDS, 4, BC, 8, SUB)
    return out5.transpose(2, 4, 0, 1, 3).reshape(BATCH, N_FIELDS, EMBED_DIM)


# speedup vs baseline: 1.1150x; 1.1150x over previous
import functools

import jax
import jax.numpy as jnp
from jax import lax
from jax.experimental import pallas as pl
from jax.experimental.pallas import tpu as pltpu
from jax.experimental.pallas import tpu_sc as plsc

VOCAB = 1000000
EMBED_DIM = 32
BATCH = 16384
N_FIELDS = 26

TOT = BATCH * N_FIELDS
NUM_CORES = 2
NUM_SUBCORES = 16
NW = NUM_CORES * NUM_SUBCORES
SUB = 128
NBLK = TOT // SUB
BLK_PER_W = NBLK // NW
BC = BATCH // SUB

PITCH = EMBED_DIM
VPITCH = EMBED_DIM + 1
SW = 128
NSTRIPE = VOCAB // SW
TAIL = VOCAB - NSTRIPE * SW
STRIPE_Q, STRIPE_R = divmod(NSTRIPE, NW)

_mesh = plsc.VectorSubcoreMesh(core_axis_name="c", subcore_axis_name="s")


@functools.partial(
    pl.kernel,
    mesh=_mesh,
    out_type=jax.ShapeDtypeStruct((VOCAB * PITCH,), jnp.float32),
    scratch_types=[
        pltpu.VMEM((EMBED_DIM, SW + 1), jnp.float32),
        pltpu.VMEM((EMBED_DIM, SW + 1), jnp.float32),
        pltpu.VMEM((SW * PITCH,), jnp.float32),
        pltpu.VMEM((SW * PITCH,), jnp.float32),
        pltpu.VMEM((TAIL * EMBED_DIM,), jnp.float32),
        pltpu.SemaphoreType.DMA,
        pltpu.SemaphoreType.DMA,
        pltpu.SemaphoreType.DMA,
        pltpu.SemaphoreType.DMA,
    ],
    compiler_params=pltpu.CompilerParams(
        use_tc_tiling_on_sc=True, needs_layout_passes=False
    ),
)
def _table_prep(w4_hbm, wtail_hbm, t_hbm, vbufa, vbufb, trowa, trowb, tailv,
                isema, isemb, osema, osemb):
    wid = lax.axis_index("s") * NUM_CORES + lax.axis_index("c")
    nst = jnp.where(wid < STRIPE_R, STRIPE_Q + 1, STRIPE_Q)
    start = wid * STRIPE_Q + jnp.minimum(wid, STRIPE_R)

    iota = lax.iota(jnp.int32, 16)

    def fire_in(s, vbuf, sem):
        for dq in range(4):
            pltpu.async_copy(
                w4_hbm.at[dq, :, pl.ds(s * SW, SW)],
                vbuf.at[pl.ds(dq * 8, 8), pl.ds(0, SW)],
                sem,
            )

    def drain_in(vbuf, sem):
        for dq in range(4):
            pltpu.make_async_copy(
                w4_hbm.at[0, :, pl.ds(0, SW)],
                vbuf.at[pl.ds(0, 8), pl.ds(0, SW)],
                sem,
            ).wait()

    def transpose(vbuf, trow):
        @plsc.parallel_loop(0, SW, step=4, unroll=4)
        def _vl_body(vl0):
            for u in range(4):
                vl = vl0 + u
                colv = jnp.full((16,), vl, jnp.int32)
                for c in range(EMBED_DIM // 16):
                    vals = plsc.load_gather(vbuf, [iota + c * 16, colv])
                    idx = iota + (vl * PITCH + c * 16)
                    plsc.store_scatter(trow, [idx], vals)

    def fire_out(s, trow, sem):
        pltpu.async_copy(
            trow, t_hbm.at[pl.ds(s * SW * PITCH, SW * PITCH)], sem
        )

    def drain_out(trow, sem):
        pltpu.make_async_copy(
            trow, t_hbm.at[pl.ds(0, SW * PITCH)], sem
        ).wait()

    def step(i, vbuf, trow, isem, osem, first):
        drain_in(vbuf, isem)
        if not first:
            drain_out(trow, osem)
        transpose(vbuf, trow)
        fire_out(start + i, trow, osem)

    fire_in(start, vbufa, isema)
    fire_in(start + 1, vbufb, isemb)

    def half(p, i0, vbuf, trow, isem, osem, first, off):
        step(i0 + off, vbuf, trow, isem, osem, first)

        @pl.when(i0 + off + 2 < nst)
        def _():
            fire_in(start + i0 + off + 2, vbuf, isem)

    half(0, 0, vbufa, trowa, isema, osema, True, 0)
    half(0, 0, vbufb, trowb, isemb, osemb, True, 1)

    def pair_body(p, carry):
        i0 = 2 * p
        half(p, i0, vbufa, trowa, isema, osema, False, 0)
        half(p, i0, vbufb, trowb, isemb, osemb, False, 1)
        return carry

    lax.fori_loop(1, STRIPE_Q // 2, pair_body, 0)

    @pl.when(wid < STRIPE_R)
    def _():
        step(STRIPE_Q, vbufa, trowa, isema, osema, False)

    drain_out(trowa, osema)
    drain_out(trowb, osemb)

    @pl.when(wid == NW - 1)
    def _():
        pltpu.sync_copy(wtail_hbm, tailv)

        def tail_body(r, carry):
            for c in range(EMBED_DIM // 16):
                vals = tailv[pl.ds(r * EMBED_DIM + c * 16, 16)]
                idx = iota + (r * PITCH + c * 16)
                plsc.store_scatter(trowa, [idx], vals)
            return carry

        lax.fori_loop(0, TAIL, tail_body, 0)
        pltpu.sync_copy(
            trowa.at[pl.ds(0, TAIL * PITCH)],
            t_hbm.at[pl.ds(NSTRIPE * SW * PITCH, TAIL * PITCH)],
        )


@functools.partial(
    pl.kernel,
    mesh=_mesh,
    out_type=jax.ShapeDtypeStruct((N_FIELDS, 4, BC, 8 * SUB), jnp.float32),
    scratch_types=[
        pltpu.VMEM((BLK_PER_W, SUB), jnp.int32),
        pltpu.VMEM((SUB, EMBED_DIM), jnp.float32),
        pltpu.VMEM((SUB, EMBED_DIM), jnp.float32),
        pltpu.VMEM((SUB, VPITCH), jnp.float32),
        pltpu.VMEM((SUB, VPITCH), jnp.float32),
        pltpu.VMEM((SUB * EMBED_DIM,), jnp.float32),
        pltpu.VMEM((SUB * EMBED_DIM,), jnp.float32),
        pltpu.SemaphoreType.DMA,
        pltpu.SemaphoreType.DMA,
        pltpu.SemaphoreType.DMA,
        pltpu.SemaphoreType.DMA,
    ],
    compiler_params=pltpu.CompilerParams(
        use_tc_tiling_on_sc=False, needs_layout_passes=False
    ),
)
def _emb_lookup(idx_hbm, table_hbm, out_hbm, idx_v, bufa, bufb, pbufa, pbufb,
                tbufa, tbufb, gsema, gsemb, osema, osemb):
    wid = lax.axis_index("s") * NUM_CORES + lax.axis_index("c")
    base = wid * BLK_PER_W

    pltpu.sync_copy(idx_hbm.at[pl.ds(base, BLK_PER_W)], idx_v)

    iota = lax.iota(jnp.int32, 16)

    def fire_gather(j, buf, sem):
        pltpu.async_copy(table_hbm.at[idx_v.at[j]], buf, sem)

    def drain_gather(buf, sem):
        pltpu.make_async_copy(table_hbm.at[pl.ds(0, SUB)], buf, sem).wait()

    def transpose(buf, pbuf, tbuf):
        @plsc.parallel_loop(0, SUB, step=4, unroll=4)
        def _bl_body(bl0):
            for u in range(4):
                bl = bl0 + u
                for c in range(EMBED_DIM // 16):
                    vals = buf[bl, pl.ds(c * 16, 16)]
                    pbuf[bl, pl.ds(c * 16, 16)] = vals

        @plsc.parallel_loop(0, EMBED_DIM, step=4, unroll=4)
        def _d_body(d0):
            for u in range(4):
                d = d0 + u
                col = jnp.full((16,), d, jnp.int32)
                for c in range(SUB // 16):
                    vals = plsc.load_gather(pbuf, [iota + c * 16, col])
                    tbuf[pl.ds(d * SUB + c * 16, 16)] = vals

    def fire_out(j, tbuf, sem):
        blk = base + j
        f = blk // BC
        bc = blk % BC
        for dq in range(4):
            pltpu.async_copy(
                tbuf.at[pl.ds(dq * 8 * SUB, 8 * SUB)],
                out_hbm.at[f, dq, bc],
                sem,
            )

    def drain_out(tbuf, sem):
        for dq in range(4):
            pltpu.make_async_copy(
                tbuf.at[pl.ds(0, 8 * SUB)], out_hbm.at[0, 0, 0], sem
            ).wait()

    fire_gather(0, bufa, gsema)

    def step(j, buf, pbuf, tbuf, gsem, osem, first, fire_next, nxt_buf,
             nxt_gsem):
        if fire_next:
            fire_gather(j + 1, nxt_buf, nxt_gsem)
        drain_gather(buf, gsem)
        if not first:
            drain_out(tbuf, osem)
        transpose(buf, pbuf, tbuf)
        fire_out(j, tbuf, osem)

    step(0, bufa, pbufa, tbufa, gsema, osema, True, True, bufb, gsemb)
    step(1, bufb, pbufb, tbufb, gsemb, osemb, True, True, bufa, gsema)

    def pair_body(p, carry):
        j0 = 2 * p
        step(j0, bufa, pbufa, tbufa, gsema, osema, False, True, bufb, gsemb)
        step(j0 + 1, bufb, pbufb, tbufb, gsemb, osemb, False, True, bufa,
             gsema)
        return carry

    lax.fori_loop(1, BLK_PER_W // 2 - 1, pair_body, 0)

    step(BLK_PER_W - 2, bufa, pbufa, tbufa, gsema, osema, False, True, bufb,
         gsemb)
    step(BLK_PER_W - 1, bufb, pbufb, tbufb, gsemb, osemb, False, False, bufa,
         gsema)

    drain_out(tbufa, osema)
    drain_out(tbufb, osemb)


def kernel(input, weight):
    w4 = jnp.transpose(weight).reshape(4, 8, VOCAB)
    wtail = weight[NSTRIPE * SW:].reshape(TAIL * EMBED_DIM)
    t1d = _table_prep(w4, wtail)
    table2 = t1d.reshape(VOCAB, PITCH)

    idx2d = jnp.transpose(input).reshape(NBLK, SUB)
    out4 = _emb_lookup(idx2d, table2)
    out5 = out4.reshape(N_FIELDS, 4, BC, 8, SUB)
    return out5.transpose(2, 4, 0, 1, 3).reshape(BATCH, N_FIELDS, EMBED_DIM)


# speedup vs baseline: 1.1417x; 1.0240x over previous
import functools

import jax
import jax.numpy as jnp
from jax import lax
from jax.experimental import pallas as pl
from jax.experimental.pallas import tpu as pltpu
from jax.experimental.pallas import tpu_sc as plsc

VOCAB = 1000000
EMBED_DIM = 32
BATCH = 16384
N_FIELDS = 26

TOT = BATCH * N_FIELDS
NUM_CORES = 2
NUM_SUBCORES = 16
NW = NUM_CORES * NUM_SUBCORES
SUB = 128
NBLK = TOT // SUB
BLK_PER_W = NBLK // NW
BC = BATCH // SUB

PITCH = EMBED_DIM
VPITCH = EMBED_DIM + 1
SW = 128
NSTRIPE = VOCAB // SW
TAIL = VOCAB - NSTRIPE * SW
STRIPE_Q, STRIPE_R = divmod(NSTRIPE, NW)

_mesh = plsc.VectorSubcoreMesh(core_axis_name="c", subcore_axis_name="s")


@functools.partial(
    pl.kernel,
    mesh=_mesh,
    out_type=jax.ShapeDtypeStruct((VOCAB * PITCH,), jnp.float32),
    scratch_types=[
        pltpu.VMEM((EMBED_DIM, SW + 1), jnp.float32),
        pltpu.VMEM((EMBED_DIM, SW + 1), jnp.float32),
        pltpu.VMEM((SW * PITCH,), jnp.float32),
        pltpu.VMEM((SW * PITCH,), jnp.float32),
        pltpu.VMEM((TAIL * EMBED_DIM,), jnp.float32),
        pltpu.SemaphoreType.DMA,
        pltpu.SemaphoreType.DMA,
        pltpu.SemaphoreType.DMA,
        pltpu.SemaphoreType.DMA,
    ],
    compiler_params=pltpu.CompilerParams(
        use_tc_tiling_on_sc=True, needs_layout_passes=False
    ),
)
def _table_prep(w4_hbm, wtail_hbm, t_hbm, vbufa, vbufb, trowa, trowb, tailv,
                isema, isemb, osema, osemb):
    wid = lax.axis_index("s") * NUM_CORES + lax.axis_index("c")
    nst = jnp.where(wid < STRIPE_R, STRIPE_Q + 1, STRIPE_Q)
    start = wid * STRIPE_Q + jnp.minimum(wid, STRIPE_R)

    iota = lax.iota(jnp.int32, 16)

    def fire_in(s, vbuf, sem):
        for dq in range(4):
            pltpu.async_copy(
                w4_hbm.at[dq, :, pl.ds(s * SW, SW)],
                vbuf.at[pl.ds(dq * 8, 8), pl.ds(0, SW)],
                sem,
            )

    def drain_in(vbuf, sem):
        for dq in range(4):
            pltpu.make_async_copy(
                w4_hbm.at[0, :, pl.ds(0, SW)],
                vbuf.at[pl.ds(0, 8), pl.ds(0, SW)],
                sem,
            ).wait()

    def transpose(vbuf, trow):
        @plsc.parallel_loop(0, SW, step=4, unroll=4)
        def _vl_body(vl0):
            for u in range(4):
                vl = vl0 + u
                colv = jnp.full((16,), vl, jnp.int32)
                for c in range(EMBED_DIM // 16):
                    vals = plsc.load_gather(vbuf, [iota + c * 16, colv])
                    trow[pl.ds(vl * PITCH + c * 16, 16)] = vals

    def fire_out(s, trow, sem):
        pltpu.async_copy(
            trow, t_hbm.at[pl.ds(s * SW * PITCH, SW * PITCH)], sem
        )

    def drain_out(trow, sem):
        pltpu.make_async_copy(
            trow, t_hbm.at[pl.ds(0, SW * PITCH)], sem
        ).wait()

    def step(i, vbuf, trow, isem, osem, first):
        drain_in(vbuf, isem)
        if not first:
            drain_out(trow, osem)
        transpose(vbuf, trow)
        fire_out(start + i, trow, osem)

    fire_in(start, vbufa, isema)
    fire_in(start + 1, vbufb, isemb)

    def half(p, i0, vbuf, trow, isem, osem, first, off):
        step(i0 + off, vbuf, trow, isem, osem, first)

        @pl.when(i0 + off + 2 < nst)
        def _():
            fire_in(start + i0 + off + 2, vbuf, isem)

    half(0, 0, vbufa, trowa, isema, osema, True, 0)
    half(0, 0, vbufb, trowb, isemb, osemb, True, 1)

    def pair_body(p, carry):
        i0 = 2 * p
        half(p, i0, vbufa, trowa, isema, osema, False, 0)
        half(p, i0, vbufb, trowb, isemb, osemb, False, 1)
        return carry

    lax.fori_loop(1, STRIPE_Q // 2, pair_body, 0)

    @pl.when(wid < STRIPE_R)
    def _():
        step(STRIPE_Q, vbufa, trowa, isema, osema, False)

    drain_out(trowa, osema)
    drain_out(trowb, osemb)

    @pl.when(wid == NW - 1)
    def _():
        pltpu.sync_copy(wtail_hbm, tailv)

        def tail_body(r, carry):
            for c in range(EMBED_DIM // 16):
                vals = tailv[pl.ds(r * EMBED_DIM + c * 16, 16)]
                idx = iota + (r * PITCH + c * 16)
                plsc.store_scatter(trowa, [idx], vals)
            return carry

        lax.fori_loop(0, TAIL, tail_body, 0)
        pltpu.sync_copy(
            trowa.at[pl.ds(0, TAIL * PITCH)],
            t_hbm.at[pl.ds(NSTRIPE * SW * PITCH, TAIL * PITCH)],
        )


@functools.partial(
    pl.kernel,
    mesh=_mesh,
    out_type=jax.ShapeDtypeStruct((N_FIELDS, 4, BC, 8 * SUB), jnp.float32),
    scratch_types=[
        pltpu.VMEM((BLK_PER_W, SUB), jnp.int32),
        pltpu.VMEM((SUB, EMBED_DIM), jnp.float32),
        pltpu.VMEM((SUB, EMBED_DIM), jnp.float32),
        pltpu.VMEM((SUB, VPITCH), jnp.float32),
        pltpu.VMEM((SUB, VPITCH), jnp.float32),
        pltpu.VMEM((SUB * EMBED_DIM,), jnp.float32),
        pltpu.VMEM((SUB * EMBED_DIM,), jnp.float32),
        pltpu.SemaphoreType.DMA,
        pltpu.SemaphoreType.DMA,
        pltpu.SemaphoreType.DMA,
        pltpu.SemaphoreType.DMA,
    ],
    compiler_params=pltpu.CompilerParams(
        use_tc_tiling_on_sc=False, needs_layout_passes=False
    ),
)
def _emb_lookup(idx_hbm, table_hbm, out_hbm, idx_v, bufa, bufb, pbufa, pbufb,
                tbufa, tbufb, gsema, gsemb, osema, osemb):
    wid = lax.axis_index("s") * NUM_CORES + lax.axis_index("c")
    base = wid * BLK_PER_W

    pltpu.sync_copy(idx_hbm.at[pl.ds(base, BLK_PER_W)], idx_v)

    iota = lax.iota(jnp.int32, 16)

    def fire_gather(j, buf, sem):
        pltpu.async_copy(table_hbm.at[idx_v.at[j]], buf, sem)

    def drain_gather(buf, sem):
        pltpu.make_async_copy(table_hbm.at[pl.ds(0, SUB)], buf, sem).wait()

    def transpose(buf, pbuf, tbuf):
        @plsc.parallel_loop(0, SUB, step=4, unroll=4)
        def _bl_body(bl0):
            for u in range(4):
                bl = bl0 + u
                for c in range(EMBED_DIM // 16):
                    vals = buf[bl, pl.ds(c * 16, 16)]
                    pbuf[bl, pl.ds(c * 16, 16)] = vals

        @plsc.parallel_loop(0, EMBED_DIM, step=4, unroll=4)
        def _d_body(d0):
            for u in range(4):
                d = d0 + u
                col = jnp.full((16,), d, jnp.int32)
                for c in range(SUB // 16):
                    vals = plsc.load_gather(pbuf, [iota + c * 16, col])
                    tbuf[pl.ds(d * SUB + c * 16, 16)] = vals

    def fire_out(j, tbuf, sem):
        blk = base + j
        f = blk // BC
        bc = blk % BC
        for dq in range(4):
            pltpu.async_copy(
                tbuf.at[pl.ds(dq * 8 * SUB, 8 * SUB)],
                out_hbm.at[f, dq, bc],
                sem,
            )

    def drain_out(tbuf, sem):
        for dq in range(4):
            pltpu.make_async_copy(
                tbuf.at[pl.ds(0, 8 * SUB)], out_hbm.at[0, 0, 0], sem
            ).wait()

    fire_gather(0, bufa, gsema)

    def step(j, buf, pbuf, tbuf, gsem, osem, first, fire_next, nxt_buf,
             nxt_gsem):
        if fire_next:
            fire_gather(j + 1, nxt_buf, nxt_gsem)
        drain_gather(buf, gsem)
        if not first:
            drain_out(tbuf, osem)
        transpose(buf, pbuf, tbuf)
        fire_out(j, tbuf, osem)

    step(0, bufa, pbufa, tbufa, gsema, osema, True, True, bufb, gsemb)
    step(1, bufb, pbufb, tbufb, gsemb, osemb, True, True, bufa, gsema)

    def pair_body(p, carry):
        j0 = 2 * p
        step(j0, bufa, pbufa, tbufa, gsema, osema, False, True, bufb, gsemb)
        step(j0 + 1, bufb, pbufb, tbufb, gsemb, osemb, False, True, bufa,
             gsema)
        return carry

    lax.fori_loop(1, BLK_PER_W // 2 - 1, pair_body, 0)

    step(BLK_PER_W - 2, bufa, pbufa, tbufa, gsema, osema, False, True, bufb,
         gsemb)
    step(BLK_PER_W - 1, bufb, pbufb, tbufb, gsemb, osemb, False, False, bufa,
         gsema)

    drain_out(tbufa, osema)
    drain_out(tbufb, osemb)


def kernel(input, weight):
    w4 = jnp.transpose(weight).reshape(4, 8, VOCAB)
    wtail = weight[NSTRIPE * SW:].reshape(TAIL * EMBED_DIM)
    t1d = _table_prep(w4, wtail)
    table2 = t1d.reshape(VOCAB, PITCH)

    idx2d = jnp.transpose(input).reshape(NBLK, SUB)
    out4 = _emb_lookup(idx2d, table2)
    out5 = out4.reshape(N_FIELDS, 4, BC, 8, SUB)
    return out5.transpose(2, 4, 0, 1, 3).reshape(BATCH, N_FIELDS, EMBED_DIM)
